# Initial kernel scaffold; baseline (speedup 1.0000x reference)
import jax
import jax.numpy as jnp
from jax.experimental import pallas as pl


def kernel(input):
    raise NotImplementedError("write your pallas kernel here")


# speedup vs baseline: 15.0348x; 15.0348x over previous
import functools

import jax
import jax.numpy as jnp
from jax.experimental import pallas as pl

_N = 32768
_BR = 8
_ITERS = 24


def _body(x_ref, o_ref):
    x = x_ref[...]
    m = jnp.max(x, axis=-1, keepdims=True)
    lo = m - 1.0
    hi = m

    def it(_, carry):
        lo, hi = carry
        mid = 0.5 * (lo + hi)
        s = jnp.sum(jnp.maximum(x - mid, 0.0), axis=-1, keepdims=True)
        pred = s > 1.0
        return jnp.where(pred, mid, lo), jnp.where(pred, hi, mid)

    lo, hi = jax.lax.fori_loop(0, _ITERS, it, (lo, hi))
    sup = x > lo
    k = jnp.sum(sup.astype(jnp.float32), axis=-1, keepdims=True)
    s = jnp.sum(jnp.where(sup, x, 0.0), axis=-1, keepdims=True)
    tau = (s - 1.0) / k
    o_ref[...] = jnp.maximum(x - tau, 0.0)


@jax.jit
def kernel(input):
    rows = input.shape[0]
    return pl.pallas_call(
        _body,
        grid=(rows // _BR,),
        in_specs=[pl.BlockSpec((_BR, _N), lambda i: (i, 0))],
        out_specs=pl.BlockSpec((_BR, _N), lambda i: (i, 0)),
        out_shape=jax.ShapeDtypeStruct(input.shape, input.dtype),
    )(input)
